# baseline (speedup 1.0000x reference)
import functools

import jax
import jax.numpy as jnp
from jax import lax
from jax.experimental import pallas as pl
from jax.experimental.pallas import tpu as pltpu
from jax.experimental.pallas import tpu_sc as plsc

_N = 10000
_E = 320000
_DIM = 64
_GAMMA = 1.0

_NC = 2
_NS = 16
_K = 128
_CH = 158
_EPT = _CH * _K
_EPAD = _NS * _EPT
_RPT = _N // _NS

_BN = 2000


def _pre_body(f_ref, km_ref, kv_ref, out_ref):
    f = f_ref[...]
    dn = (((1,), (0,)), ((), ()))
    m = lax.dot_general(f[:, :_DIM], km_ref[...], dn,
                        precision=lax.Precision.HIGHEST,
                        preferred_element_type=jnp.float32)
    v = lax.dot_general(f[:, _DIM:], kv_ref[...], dn,
                        precision=lax.Precision.HIGHEST,
                        preferred_element_type=jnp.float32)
    m = jnp.where(m > 0.0, m, jnp.exp(m) - 1.0)
    v = jnp.maximum(v, 0.0)
    att = jnp.exp(-_GAMMA * v)
    out_ref[0] = m * att
    out_ref[1] = v * att * att


def _post_body(agg_ref, noise_ref, out_ref):
    out_ref[...] = agg_ref[0] + jnp.sqrt(agg_ref[1] + 1e-8) * noise_ref[...]


_sc_mesh = plsc.VectorSubcoreMesh(
    core_axis_name="c", subcore_axis_name="s", num_cores=_NC, num_subcores=_NS
)


@functools.partial(
    pl.kernel,
    out_type=jax.ShapeDtypeStruct((_NC, _NS, _RPT, _DIM), jnp.float32),
    mesh=_sc_mesh,
    compiler_params=pltpu.CompilerParams(
        needs_layout_passes=False, use_tc_tiling_on_sc=False),
    scratch_types=[
        pltpu.VMEM((_CH, _K), jnp.int32),
        pltpu.VMEM((_CH, _K), jnp.int32),
        pltpu.VMEM((_EPT,), jnp.float32),
        pltpu.VMEM((_K, _DIM), jnp.float32),
        pltpu.VMEM((_K, _DIM), jnp.float32),
        pltpu.VMEM((_K, _DIM), jnp.float32),
        pltpu.VMEM_SHARED((_N, _DIM), jnp.float32),
        pltpu.SemaphoreType.DMA,
        pltpu.SemaphoreType.DMA,
    ],
)
def _edge_kernel(x2_hbm, src_hbm, dst_hbm, w_hbm, out_hbm,
                 src_v, dst_v, w_v, ga_v, gb_v, sc_v, agg_sh, gsem0, gsem1):
    c = lax.axis_index("c")
    s = lax.axis_index("s")

    pltpu.sync_copy(src_hbm.at[c, s], src_v)
    pltpu.sync_copy(dst_hbm.at[s], dst_v)
    pltpu.sync_copy(w_hbm.at[c, s], w_v)

    def zrow(e, carry):
        for q in range(_DIM // 16):
            ga_v[e, pl.ds(q * 16, 16)] = jnp.zeros((16,), jnp.float32)
        return carry

    lax.fori_loop(0, _K, zrow, 0)

    base = s * _RPT
    n_full = _RPT // _K
    rem = _RPT - n_full * _K

    def zcp(i, carry):
        pltpu.sync_copy(ga_v, agg_sh.at[pl.ds(base + i * _K, _K)])
        return carry

    lax.fori_loop(0, n_full, zcp, 0)
    if rem:
        pltpu.sync_copy(ga_v.at[pl.ds(0, rem)],
                        agg_sh.at[pl.ds(base + n_full * _K, rem)])
    plsc.subcore_barrier()

    def scale(gbuf, j):
        wbase = j * _K

        def group(g, c2):
            for e in range(16):
                eidx = g * 16 + e
                wb = plsc.load_gather(
                    w_v, [jnp.full((16,), wbase + eidx, jnp.int32)])
                for q in range(_DIM // 16):
                    sl = pl.ds(q * 16, 16)
                    sc_v[eidx, sl] = gbuf[eidx, sl] * wb
            return c2

        lax.fori_loop(0, _K // 16, group, 0)

    pltpu.async_copy(x2_hbm.at[src_v.at[0]], ga_v, gsem0)
    pltpu.async_copy(x2_hbm.at[src_v.at[1]], gb_v, gsem1)

    bufs = ((ga_v, gsem0), (gb_v, gsem1))

    def pair(i, carry):
        j0 = 2 * i
        for b, (gbuf, gsem) in enumerate(bufs):
            j = j0 + b
            pltpu.make_async_copy(x2_hbm.at[src_v.at[j]], gbuf, gsem).wait()
            scale(gbuf, j)
            pltpu.sync_copy(sc_v, agg_sh.at[dst_v.at[j]], add=True)
            @pl.when(j + 2 < _CH)
            def _refill():
                pltpu.async_copy(x2_hbm.at[src_v.at[j + 2]], gbuf, gsem)
        return carry

    lax.fori_loop(0, _CH // 2, pair, 0)
    plsc.subcore_barrier()

    pltpu.sync_copy(agg_sh.at[pl.ds(base, _RPT)], out_hbm.at[c, s])


def kernel(features, edge_index, adj0_weight, adj1_weight,
           kernel_mean, kernel_var, noise):
    x2 = pl.pallas_call(
        _pre_body,
        grid=(_N // _BN,),
        in_specs=[
            pl.BlockSpec((_BN, 2 * _DIM), lambda i: (i, 0)),
            pl.BlockSpec((_DIM, _DIM), lambda i: (0, 0)),
            pl.BlockSpec((_DIM, _DIM), lambda i: (0, 0)),
        ],
        out_specs=pl.BlockSpec((2, _BN, _DIM), lambda i: (0, i, 0)),
        out_shape=jax.ShapeDtypeStruct((2, _N, _DIM), jnp.float32),
    )(features, kernel_mean, kernel_var)
    x2f = x2.reshape(2 * _N, _DIM)

    dst = edge_index[0]
    src = edge_index[1]
    pad = _EPAD - _E
    srcp = jnp.pad(src, (0, pad)).reshape(_NS, _CH, _K)
    dstp = jnp.pad(dst, (0, pad)).reshape(_NS, _CH, _K)
    src2 = srcp[None] + (jnp.arange(_NC, dtype=jnp.int32) * _N)[:, None, None, None]
    w2 = jnp.stack([
        jnp.pad(adj0_weight, (0, pad)),
        jnp.pad(adj1_weight, (0, pad)),
    ]).reshape(_NC, _NS, _EPT)

    agg = _edge_kernel(x2f, src2, dstp, w2).reshape(_NC, _N, _DIM)

    out = pl.pallas_call(
        _post_body,
        grid=(_N // _BN,),
        in_specs=[
            pl.BlockSpec((2, _BN, _DIM), lambda i: (0, i, 0)),
            pl.BlockSpec((_BN, _DIM), lambda i: (i, 0)),
        ],
        out_specs=pl.BlockSpec((_BN, _DIM), lambda i: (i, 0)),
        out_shape=jax.ShapeDtypeStruct((_N, _DIM), jnp.float32),
    )(agg, noise)
    return out


# speedup vs baseline: 1.1331x; 1.1331x over previous
import functools

import jax
import jax.numpy as jnp
from jax import lax
from jax.experimental import pallas as pl
from jax.experimental.pallas import tpu as pltpu
from jax.experimental.pallas import tpu_sc as plsc

_N = 10000
_E = 320000
_DIM = 64
_GAMMA = 1.0

_NC = 2
_NS = 16
_K = 128
_CH = 80
_H = 64
_EPT = _CH * _K
_EPAD = _NC * _NS * _EPT
_RPT = _N // _NS

_BN = 2000


def _pre_body(f_ref, km_ref, kv_ref, out_ref):
    f = f_ref[...]
    dn = (((1,), (0,)), ((), ()))
    m = lax.dot_general(f[:, :_DIM], km_ref[...], dn,
                        precision=lax.Precision.HIGHEST,
                        preferred_element_type=jnp.float32)
    v = lax.dot_general(f[:, _DIM:], kv_ref[...], dn,
                        precision=lax.Precision.HIGHEST,
                        preferred_element_type=jnp.float32)
    m = jnp.where(m > 0.0, m, jnp.exp(m) - 1.0)
    v = jnp.maximum(v, 0.0)
    att = jnp.exp(-_GAMMA * v)
    out_ref[:, :_DIM] = (m * att).astype(jnp.bfloat16)
    out_ref[:, _DIM:] = (v * att * att).astype(jnp.bfloat16)


def _post_body(agg_ref, noise_ref, out_ref):
    su = agg_ref[0] + agg_ref[1]
    out_ref[...] = su[:, :_DIM] + jnp.sqrt(su[:, _DIM:] + 1e-8) * noise_ref[...]


_sc_mesh = plsc.VectorSubcoreMesh(
    core_axis_name="c", subcore_axis_name="s", num_cores=_NC, num_subcores=_NS
)


@functools.partial(
    pl.kernel,
    out_type=jax.ShapeDtypeStruct((_NC, _NS, _RPT, 2 * _DIM), jnp.float32),
    mesh=_sc_mesh,
    compiler_params=pltpu.CompilerParams(
        needs_layout_passes=False, use_tc_tiling_on_sc=False),
    scratch_types=[
        pltpu.VMEM((_CH, _K), jnp.int16),
        pltpu.VMEM((_CH, _K), jnp.int16),
        pltpu.VMEM((_EPT,), jnp.int32),
        pltpu.VMEM((2, _K), jnp.int32),
        pltpu.VMEM((2, _H), jnp.int32),
        pltpu.VMEM((_K, 2 * _DIM), jnp.bfloat16),
        pltpu.VMEM((_K, 2 * _DIM), jnp.bfloat16),
        pltpu.VMEM((_H, 2 * _DIM), jnp.float32),
        pltpu.VMEM_SHARED((_N, 2 * _DIM), jnp.float32),
        pltpu.SemaphoreType.DMA,
        pltpu.SemaphoreType.DMA,
    ],
)
def _edge_kernel(xt_hbm, src_hbm, dst_hbm, w_hbm, out_hbm,
                 src16_v, dst16_v, w_v, src32_v, dst32_v,
                 ga_v, gb_v, sc_v, agg_sh, gsem0, gsem1):
    c = lax.axis_index("c")
    s = lax.axis_index("s")

    pltpu.sync_copy(src_hbm.at[c, s], src16_v)
    pltpu.sync_copy(dst_hbm.at[c, s], dst16_v)
    pltpu.sync_copy(w_hbm.at[c, s], w_v)

    def zrow(e, carry):
        for q in range(2 * _DIM // 16):
            sc_v[e, pl.ds(q * 16, 16)] = jnp.zeros((16,), jnp.float32)
        return carry

    lax.fori_loop(0, _H, zrow, 0)

    base = s * _RPT
    n_full = _RPT // _H
    rem = _RPT - n_full * _H

    def zcp(i, carry):
        pltpu.sync_copy(sc_v, agg_sh.at[pl.ds(base + i * _H, _H)])
        return carry

    lax.fori_loop(0, n_full, zcp, 0)
    if rem:
        pltpu.sync_copy(sc_v.at[pl.ds(0, rem)],
                        agg_sh.at[pl.ds(base + n_full * _H, rem)])
    plsc.subcore_barrier()

    def fill_src(b, j):
        for d in range(_K // 32):
            ab = src16_v[j, pl.ds(d * 32, 32)]
            lo, hi = plsc.unpack(ab, format=plsc.PackFormat.INTERLEAVED)
            src32_v[b, pl.ds(d * 32, 16)] = lo
            src32_v[b, pl.ds(d * 32 + 16, 16)] = hi

    def fill_dst(h, j):
        for d in range(_H // 32):
            ab = dst16_v[j, pl.ds(h * _H + d * 32, 32)]
            lo, hi = plsc.unpack(ab, format=plsc.PackFormat.INTERLEAVED)
            dst32_v[h, pl.ds(d * 32, 16)] = lo
            dst32_v[h, pl.ds(d * 32 + 16, 16)] = hi

    def gather(b, gbuf, gsem):
        return pltpu.make_async_copy(xt_hbm.at[src32_v.at[b]], gbuf, gsem)

    _MASKHI = jnp.int32(-65536)

    def scale_half(gbuf, j, h):
        wbase = j * _K + h * _H

        def group(g, c2):
            for e in range(16):
                le = g * 16 + e
                wpair = plsc.load_gather(
                    w_v, [jnp.full((16,), wbase + le, jnp.int32)])
                w0 = plsc.bitcast(lax.shift_left(wpair, 16), jnp.float32)
                w1 = plsc.bitcast(wpair & _MASKHI, jnp.float32)
                for d in range(2 * _DIM // 32):
                    wb = w0 if d < 2 else w1
                    ab = gbuf[h * _H + le, pl.ds(d * 32, 32)]
                    lo, hi = plsc.unpack(
                        ab, format=plsc.PackFormat.INTERLEAVED)
                    sc_v[le, pl.ds(d * 32, 16)] = lo * wb
                    sc_v[le, pl.ds(d * 32 + 16, 16)] = hi * wb
            return c2

        lax.fori_loop(0, _H // 16, group, 0)

    fill_src(0, 0)
    gather(0, ga_v, gsem0).start()
    fill_src(1, 1)
    gather(1, gb_v, gsem1).start()

    bufs = ((ga_v, gsem0), (gb_v, gsem1))

    def pair(i, carry):
        j0 = 2 * i
        for b, (gbuf, gsem) in enumerate(bufs):
            j = j0 + b
            gather(b, gbuf, gsem).wait()
            for h in range(2):
                fill_dst(h, j)
                scale_half(gbuf, j, h)
                pltpu.sync_copy(sc_v, agg_sh.at[dst32_v.at[h]], add=True)
            @pl.when(j + 2 < _CH)
            def _refill():
                fill_src(b, j + 2)
                gather(b, gbuf, gsem).start()
        return carry

    lax.fori_loop(0, _CH // 2, pair, 0)
    plsc.subcore_barrier()

    pltpu.sync_copy(agg_sh.at[pl.ds(base, _RPT)], out_hbm.at[c, s])


def _interleave16(x):
    lead = x.shape[:-1]
    n = x.shape[-1]
    y = x.reshape(lead + (n // 32, 2, 16))
    return jnp.swapaxes(y, -1, -2).reshape(lead + (n,))


def _bf16_bits(w):
    return lax.bitcast_convert_type(
        w.astype(jnp.bfloat16), jnp.uint16).astype(jnp.uint32)


def kernel(features, edge_index, adj0_weight, adj1_weight,
           kernel_mean, kernel_var, noise):
    xt = pl.pallas_call(
        _pre_body,
        grid=(_N // _BN,),
        in_specs=[
            pl.BlockSpec((_BN, 2 * _DIM), lambda i: (i, 0)),
            pl.BlockSpec((_DIM, _DIM), lambda i: (0, 0)),
            pl.BlockSpec((_DIM, _DIM), lambda i: (0, 0)),
        ],
        out_specs=pl.BlockSpec((_BN, 2 * _DIM), lambda i: (i, 0)),
        out_shape=jax.ShapeDtypeStruct((_N, 2 * _DIM), jnp.bfloat16),
    )(features, kernel_mean, kernel_var)
    xtp = _interleave16(xt)

    dst = edge_index[0]
    src = edge_index[1]
    pad = _EPAD - _E
    srcp = jnp.pad(src, (0, pad)).reshape(_NC, _NS, _CH, _K)
    dstp = jnp.pad(dst, (0, pad)).reshape(_NC, _NS, _CH, _K)
    src16 = _interleave16(srcp.astype(jnp.int16))
    dst16 = _interleave16(dstp.astype(jnp.int16))
    w0b = _bf16_bits(jnp.pad(adj0_weight, (0, pad)))
    w1b = _bf16_bits(jnp.pad(adj1_weight, (0, pad)))
    wpair = lax.bitcast_convert_type(
        w0b | (w1b << 16), jnp.int32).reshape(_NC, _NS, _EPT)

    agg = _edge_kernel(xtp, src16, dst16, wpair).reshape(_NC, _N, 2 * _DIM)

    out = pl.pallas_call(
        _post_body,
        grid=(_N // _BN,),
        in_specs=[
            pl.BlockSpec((2, _BN, 2 * _DIM), lambda i: (0, i, 0)),
            pl.BlockSpec((_BN, _DIM), lambda i: (i, 0)),
        ],
        out_specs=pl.BlockSpec((_BN, _DIM), lambda i: (i, 0)),
        out_shape=jax.ShapeDtypeStruct((_N, _DIM), jnp.float32),
    )(agg, noise)
    return out
